# Initial kernel scaffold; baseline (speedup 1.0000x reference)
import jax
import jax.numpy as jnp
from jax.experimental import pallas as pl


def kernel(feat_0, feat_1, feat_2, feat_3, feat_4, feat_5, feat_6, feat_7, W_0, W_1, W_2, W_3, W_4, W_5, W_6, W_7):
    raise NotImplementedError("write your pallas kernel here")


# speedup vs baseline: 1.4282x; 1.4282x over previous
import functools

import jax
import jax.numpy as jnp
from jax import lax
from jax.experimental import pallas as pl
from jax.experimental.pallas import tpu as pltpu
from jax.experimental.pallas import tpu_sc as plsc

NUM_FEATURES = 8
VOCAB = 100000
D = 64
B = 4096
BAG = 20

NC = 2
NS = 16
NW = NC * NS
BAGS_PER_W = B // NW
CHUNK = 32
NCHUNK = BAGS_PER_W // CHUNK
ROWS = CHUNK * BAG
SEG = 128
NSEG = ROWS // SEG
LANES = 16
NG = D // LANES


def _make_kernel():
    mesh = plsc.VectorSubcoreMesh(core_axis_name="c", subcore_axis_name="s")

    @functools.partial(
        pl.kernel,
        mesh=mesh,
        compiler_params=pltpu.CompilerParams(use_tc_tiling_on_sc=False),
        out_type=tuple(
            jax.ShapeDtypeStruct((B, D), jnp.float32) for _ in range(NUM_FEATURES)
        ),
        scratch_types=[
            pltpu.VMEM((ROWS,), jnp.int32),
            pltpu.VMEM((ROWS, D), jnp.float32),
            pltpu.VMEM((CHUNK, D), jnp.float32),
            pltpu.SemaphoreType.DMA,
        ],
    )
    def ebag(idx0, idx1, idx2, idx3, idx4, idx5, idx6, idx7,
             W0, W1, W2, W3, W4, W5, W6, W7,
             o0, o1, o2, o3, o4, o5, o6, o7,
             idx_v, rows_v, out_v, sem):
        idxs = (idx0, idx1, idx2, idx3, idx4, idx5, idx6, idx7)
        Ws = (W0, W1, W2, W3, W4, W5, W6, W7)
        outs = (o0, o1, o2, o3, o4, o5, o6, o7)
        wid = lax.axis_index("s") * NC + lax.axis_index("c")
        inv = jnp.full((LANES,), 1.0 / BAG, dtype=jnp.float32)

        for f in range(NUM_FEATURES):
            idx_hbm, W_hbm, out_hbm = idxs[f], Ws[f], outs[f]

            def chunk_body(c, _, idx_hbm=idx_hbm, W_hbm=W_hbm, out_hbm=out_hbm):
                bag0 = wid * BAGS_PER_W + c * CHUNK
                ioff = bag0 * BAG
                pltpu.sync_copy(idx_hbm.at[pl.ds(ioff, ROWS)], idx_v)
                copies = []
                for j in range(NSEG):
                    copies.append(pltpu.async_copy(
                        W_hbm.at[idx_v.at[pl.ds(j * SEG, SEG)]],
                        rows_v.at[pl.ds(j * SEG, SEG), :],
                        sem,
                    ))
                for cp in copies:
                    cp.wait()

                def bag_body(b, _):
                    rb = b * BAG
                    for g in range(NG):
                        sl = pl.ds(g * LANES, LANES)
                        acc = rows_v[rb, sl]
                        for j in range(1, BAG):
                            acc = acc + rows_v[rb + j, sl]
                        out_v[b, sl] = acc * inv
                    return 0

                lax.fori_loop(0, CHUNK, bag_body, 0)
                pltpu.sync_copy(out_v, out_hbm.at[pl.ds(bag0, CHUNK), :])
                return 0

            lax.fori_loop(0, NCHUNK, chunk_body, 0)

    return ebag


_EBAG = _make_kernel()


def kernel(feat_0, feat_1, feat_2, feat_3, feat_4, feat_5, feat_6, feat_7,
           W_0, W_1, W_2, W_3, W_4, W_5, W_6, W_7):
    feats = (feat_0, feat_1, feat_2, feat_3, feat_4, feat_5, feat_6, feat_7)
    idxs = tuple(f.reshape(B * BAG) for f in feats)
    return _EBAG(*idxs, W_0, W_1, W_2, W_3, W_4, W_5, W_6, W_7)


# speedup vs baseline: 1.6140x; 1.1300x over previous
import functools

import jax
import jax.numpy as jnp
from jax import lax
from jax.experimental import pallas as pl
from jax.experimental.pallas import tpu as pltpu
from jax.experimental.pallas import tpu_sc as plsc

NUM_FEATURES = 8
VOCAB = 100000
D = 64
B = 4096
BAG = 20

NC = 2
NS = 16
NW = NC * NS
BAGS_PER_W = B // NW
CHUNK = 32
NCHUNK = BAGS_PER_W // CHUNK
ROWS = CHUNK * BAG
SEG = 128
NSEG = ROWS // SEG
LANES = 16
NG = D // LANES
IDX_PER_W = BAGS_PER_W * BAG


def _make_kernel():
    mesh = plsc.VectorSubcoreMesh(core_axis_name="c", subcore_axis_name="s")

    @functools.partial(
        pl.kernel,
        mesh=mesh,
        compiler_params=pltpu.CompilerParams(use_tc_tiling_on_sc=False),
        out_type=tuple(
            jax.ShapeDtypeStruct((B, D), jnp.float32) for _ in range(NUM_FEATURES)
        ),
        scratch_types=[
            pltpu.VMEM((NUM_FEATURES, IDX_PER_W), jnp.int32),
            pltpu.VMEM((2, ROWS, D), jnp.float32),
            pltpu.VMEM((CHUNK, D), jnp.float32),
            pltpu.SemaphoreType.DMA,
            pltpu.SemaphoreType.DMA,
        ],
    )
    def ebag(idx0, idx1, idx2, idx3, idx4, idx5, idx6, idx7,
             W0, W1, W2, W3, W4, W5, W6, W7,
             o0, o1, o2, o3, o4, o5, o6, o7,
             idx_v, rows_v, out_v, gsem0, gsem1):
        idxs = (idx0, idx1, idx2, idx3, idx4, idx5, idx6, idx7)
        Ws = (W0, W1, W2, W3, W4, W5, W6, W7)
        outs = (o0, o1, o2, o3, o4, o5, o6, o7)
        gsems = (gsem0, gsem1)
        wid = lax.axis_index("s") * NC + lax.axis_index("c")
        inv = jnp.full((LANES,), 1.0 / BAG, dtype=jnp.float32)

        for f in range(NUM_FEATURES):
            pltpu.sync_copy(idxs[f].at[pl.ds(wid * IDX_PER_W, IDX_PER_W)],
                            idx_v.at[f])

        def fire(f, c, buf):
            for j in range(NSEG):
                pltpu.async_copy(
                    Ws[f].at[idx_v.at[f, pl.ds(c * ROWS + j * SEG, SEG)]],
                    rows_v.at[buf, pl.ds(j * SEG, SEG), :],
                    gsems[buf],
                )

        def drain(f, c, buf):
            for j in range(NSEG):
                pltpu.make_async_copy(
                    Ws[f].at[idx_v.at[f, pl.ds(c * ROWS + j * SEG, SEG)]],
                    rows_v.at[buf, pl.ds(j * SEG, SEG), :],
                    gsems[buf],
                ).wait()

        fire(0, 0, 0)

        for f in range(NUM_FEATURES):
            W_hbm, out_hbm = Ws[f], outs[f]

            def chunk_body(c, _, f=f, W_hbm=W_hbm, out_hbm=out_hbm):
                buf = lax.rem(c, 2)
                nbuf = 1 - buf

                @pl.when(jnp.logical_and(c + 1 < NCHUNK, nbuf == 0))
                def _():
                    fire(f, c + 1, 0)

                @pl.when(jnp.logical_and(c + 1 < NCHUNK, nbuf == 1))
                def _():
                    fire(f, c + 1, 1)

                if f + 1 < NUM_FEATURES:
                    @pl.when(jnp.logical_and(c + 1 == NCHUNK, nbuf == 0))
                    def _():
                        fire(f + 1, 0, 0)

                    @pl.when(jnp.logical_and(c + 1 == NCHUNK, nbuf == 1))
                    def _():
                        fire(f + 1, 0, 1)

                @pl.when(buf == 0)
                def _():
                    drain(f, c, 0)

                @pl.when(buf == 1)
                def _():
                    drain(f, c, 1)

                def bag_body(b, _):
                    rb = b * BAG
                    for g in range(NG):
                        sl = pl.ds(g * LANES, LANES)
                        acc = rows_v[buf, rb, sl]
                        for j in range(1, BAG):
                            acc = acc + rows_v[buf, rb + j, sl]
                        out_v[b, sl] = acc * inv
                    return 0

                lax.fori_loop(0, CHUNK, bag_body, 0)
                bag0 = wid * BAGS_PER_W + c * CHUNK
                pltpu.sync_copy(out_v, out_hbm.at[pl.ds(bag0, CHUNK), :])
                return 0

            lax.fori_loop(0, NCHUNK, chunk_body, 0)

    return ebag


_EBAG = _make_kernel()


def kernel(feat_0, feat_1, feat_2, feat_3, feat_4, feat_5, feat_6, feat_7,
           W_0, W_1, W_2, W_3, W_4, W_5, W_6, W_7):
    feats = (feat_0, feat_1, feat_2, feat_3, feat_4, feat_5, feat_6, feat_7)
    idxs = tuple(f.reshape(B * BAG) for f in feats)
    return _EBAG(*idxs, W_0, W_1, W_2, W_3, W_4, W_5, W_6, W_7)


# speedup vs baseline: 1.8149x; 1.1245x over previous
import functools

import jax
import jax.numpy as jnp
from jax import lax
from jax.experimental import pallas as pl
from jax.experimental.pallas import tpu as pltpu
from jax.experimental.pallas import tpu_sc as plsc

NUM_FEATURES = 8
VOCAB = 100000
D = 64
B = 4096
BAG = 20

NC = 2
NS = 16
NW = NC * NS
BPW = B // NW
LANES = 16
NG = D // LANES
IDX_PER_W = BPW * BAG


def _make_kernel():
    mesh = plsc.VectorSubcoreMesh(core_axis_name="c", subcore_axis_name="s")

    @functools.partial(
        pl.kernel,
        mesh=mesh,
        compiler_params=pltpu.CompilerParams(use_tc_tiling_on_sc=False),
        out_type=tuple(
            jax.ShapeDtypeStruct((B, D), jnp.float32) for _ in range(NUM_FEATURES)
        ),
        scratch_types=[
            pltpu.VMEM((NUM_FEATURES, IDX_PER_W), jnp.int32),
            pltpu.VMEM((2, BPW, D), jnp.float32),
            pltpu.VMEM((2, BPW, D), jnp.float32),
            pltpu.SemaphoreType.DMA,
            pltpu.SemaphoreType.DMA,
            pltpu.SemaphoreType.DMA,
            pltpu.SemaphoreType.DMA,
        ],
    )
    def ebag(idx0, idx1, idx2, idx3, idx4, idx5, idx6, idx7,
             W0, W1, W2, W3, W4, W5, W6, W7,
             o0, o1, o2, o3, o4, o5, o6, o7,
             idx_v, acc_v, out_v, gsem0, gsem1, osem0, osem1):
        idxs = (idx0, idx1, idx2, idx3, idx4, idx5, idx6, idx7)
        Ws = (W0, W1, W2, W3, W4, W5, W6, W7)
        outs = (o0, o1, o2, o3, o4, o5, o6, o7)
        gsems = (gsem0, gsem1)
        osems = (osem0, osem1)
        wid = lax.axis_index("s") * NC + lax.axis_index("c")
        inv = jnp.full((LANES,), 1.0 / BAG, dtype=jnp.float32)
        zero = jnp.zeros((LANES,), dtype=jnp.float32)

        for f in range(NUM_FEATURES):
            pltpu.sync_copy(idxs[f].at[pl.ds(wid * IDX_PER_W, IDX_PER_W)],
                            idx_v.at[f])

        def zero_body(b, _):
            for g in range(NG):
                sl = pl.ds(g * LANES, LANES)
                acc_v[0, b, sl] = zero
                acc_v[1, b, sl] = zero
            return 0
        lax.fori_loop(0, BPW, zero_body, 0)

        def fire(f, buf):
            return [
                pltpu.async_copy(
                    Ws[f].at[idx_v.at[f, pl.ds(j * BPW, BPW)]],
                    acc_v.at[buf],
                    gsems[buf],
                    add=True,
                )
                for j in range(BAG)
            ]

        pending = {0: fire(0, 0)}
        out_pending = {}

        for f in range(NUM_FEATURES):
            buf = f % 2
            if f + 1 < NUM_FEATURES:
                pending[f + 1] = fire(f + 1, 1 - buf)
            for cp in pending.pop(f):
                cp.wait()
            if f - 2 in out_pending:
                out_pending.pop(f - 2).wait()

            def scale_body(b, _, buf=buf):
                for g in range(NG):
                    sl = pl.ds(g * LANES, LANES)
                    out_v[buf, b, sl] = acc_v[buf, b, sl] * inv
                    acc_v[buf, b, sl] = zero
                return 0
            lax.fori_loop(0, BPW, scale_body, 0)

            out_pending[f] = pltpu.async_copy(
                out_v.at[buf],
                outs[f].at[pl.ds(wid * BPW, BPW), :],
                osems[buf],
            )

        for f in sorted(out_pending):
            out_pending.pop(f).wait()

    return ebag


_EBAG = _make_kernel()


def kernel(feat_0, feat_1, feat_2, feat_3, feat_4, feat_5, feat_6, feat_7,
           W_0, W_1, W_2, W_3, W_4, W_5, W_6, W_7):
    feats = (feat_0, feat_1, feat_2, feat_3, feat_4, feat_5, feat_6, feat_7)
    idxs = tuple(
        f.reshape(NW, BPW, BAG).transpose(0, 2, 1).reshape(B * BAG)
        for f in feats
    )
    return _EBAG(*idxs, W_0, W_1, W_2, W_3, W_4, W_5, W_6, W_7)
